# baseline (speedup 1.0000x reference)
import functools
import math

import jax
import jax.numpy as jnp
from jax import lax
from jax.experimental import pallas as pl
from jax.experimental.pallas import tpu as pltpu
from jax.experimental.pallas import tpu_sc as plsc

C = 100
ALLNUM = 100000
DIM = 64
B = 1024
TOPK = 5000
SEG = 1024
NPAD = C * SEG
CT = 2048
BR = 32
CPAD = 128
INT_MIN = -2147483648
NW = 32
RPW = B // NW
CHK = 64
NCHK = NPAD // CHK
NCHK_PAD = 2048
CAPC = 192

_DN = lax.GatherDimensionNumbers(
    offset_dims=(), collapsed_slice_dims=(0,), start_index_map=(0,))


def _key_from_boosted(boosted):
    b = lax.bitcast_convert_type(boosted, jnp.int32)
    return jnp.where(b >= 0, b, INT_MIN - b)


def _val_from_key(u):
    b = jnp.where(u >= 0, u, INT_MIN - u)
    return lax.bitcast_convert_type(b, jnp.float32)


def _k1_body(x_ref, p_ref, tgt_ref, u_ref, possum_ref):
    cb = pl.program_id(0)
    pt = p_ref[...]
    n2 = jnp.sum(pt * pt, axis=0, keepdims=True)
    invn = 1.0 / jnp.maximum(jnp.sqrt(n2), 1e-12)
    sim = jnp.dot(x_ref[...], pt,
                  preferred_element_type=jnp.float32) * invn
    j = lax.broadcasted_iota(jnp.int32, (1, CT), 1)
    cls = cb * (CT // SEG) + (j // SEG)
    ispad = (j % SEG) >= (ALLNUM // C)
    tgt = tgt_ref[...]
    pos = (cls == tgt) & jnp.logical_not(ispad)
    boosted = sim + 1000.0 * pos.astype(jnp.float32)
    u = _key_from_boosted(boosted)
    u_ref[...] = jnp.where(ispad, INT_MIN, u)

    contrib = jnp.sum(jnp.where(pos, sim, 0.0), axis=1, keepdims=True)

    @pl.when(cb == 0)
    def _():
        possum_ref[...] = jnp.zeros_like(possum_ref)

    possum_ref[...] += contrib


def _k2a_body(u_ref, tgt_ref, lob_ref, need_ref, defsum_ref, bc_ref):
    u = u_ref[...]

    def body(_, carry):
        lo, hi, cnt_hi = carry
        mid = lo + ((hi - lo) >> 1)
        cnt = jnp.sum((u >= (mid << 16)).astype(jnp.int32), axis=1,
                      keepdims=True)
        pred = cnt >= TOPK
        lo = jnp.where(pred, mid, lo)
        hi = jnp.where(pred, hi, mid)
        cnt_hi = jnp.where(pred, cnt_hi, cnt)
        return lo, hi, cnt_hi

    lo0 = jnp.full((BR, 1), -32768, jnp.int32)
    hi0 = jnp.full((BR, 1), 32768, jnp.int32)
    ch0 = jnp.zeros((BR, 1), jnp.int32)
    th16, _, cnt_above = lax.fori_loop(0, 16, body, (lo0, hi0, ch0))

    lob = th16 << 16
    hib = (th16 + 1) << 16
    lob_ref[...] = lob + jnp.zeros((BR, CPAD), jnp.int32)
    need_ref[...] = (TOPK - cnt_above) + jnp.zeros((BR, CPAD), jnp.int32)

    j = lax.broadcasted_iota(jnp.int32, (1, NPAD), 1)
    cls = j // SEG
    tgt = tgt_ref[...]
    seldef = (u >= hib) & (cls != tgt)
    vals = jnp.where(seldef, _val_from_key(u), 0.0)
    dsum = jnp.sum(vals.reshape(BR, C, SEG), axis=2)
    defsum_ref[...] = jnp.concatenate(
        [dsum, jnp.zeros((BR, CPAD - C), jnp.float32)], axis=1)

    inb = (u >= lob) & (u < hib)
    mi = jnp.where(inb, 1, 0).astype(jnp.int32)
    bc = jnp.sum(mi.reshape(BR, NCHK, CHK), axis=2)
    bc_ref[...] = jnp.concatenate(
        [bc, jnp.zeros((BR, NCHK_PAD - NCHK), jnp.int32)], axis=1)


def _bfly_sum(x):
    idx = lax.iota(jnp.int32, 16)
    for s in (8, 4, 2, 1):
        perm = idx ^ s
        x = x + lax.gather(x, perm[:, None], _DN, slice_sizes=(1,),
                           mode=lax.GatherScatterMode.PROMISE_IN_BOUNDS)
    return x[0]


def _sc_body(u_hbm, lob_hbm, need_hbm, bc_hbm, corr_hbm,
             u_row, cand_v, chid, bc_buf, lob_buf, need_buf, cls_sum,
             cls_smem):
    wid = lax.axis_index("s") * 2 + lax.axis_index("c")
    base = wid * RPW
    lane = lax.iota(jnp.int32, 16)

    def row_body(i, carry):
        r = base + i
        pltpu.sync_copy(u_hbm.at[pl.ds(r * NPAD, NPAD)], u_row)
        pltpu.sync_copy(bc_hbm.at[pl.ds(r * NCHK_PAD, NCHK)], bc_buf)
        pltpu.sync_copy(lob_hbm.at[pl.ds(r * CPAD, 16)], lob_buf)
        pltpu.sync_copy(need_hbm.at[pl.ds(r * CPAD, 16)], need_buf)
        lob_v = lob_buf[pl.ds(0, 16)]
        lob_s = lob_v[0]
        need_s = need_buf[pl.ds(0, 16)][0]
        width_v = jnp.full((16,), 65536, jnp.uint32)
        fill_v = jnp.full((16,), INT_MIN, jnp.int32)
        zero_v = jnp.full((16,), 0, jnp.int32)
        one_v = jnp.full((16,), 1, jnp.int32)
        zf = jnp.zeros((16,), jnp.float32)
        for t in range(CPAD):
            cls_smem[t] = 0.0

        def grp_body(g, nh):
            cv = bc_buf[pl.ds(g * 16, 16)]

            def one(j, nh):
                c_s = cv[j]

                def hit(nh):
                    slot = jnp.minimum(nh, CAPC - 1)
                    ck = g * 16 + j
                    for s4 in range(CHK // 16):
                        v = u_row[pl.ds(ck * CHK + s4 * 16, 16)]
                        d = lax.bitcast_convert_type(v - lob_v, jnp.uint32)
                        m = d < width_v
                        cand_v[pl.ds((slot * 4 + s4) * 16, 16)] = \
                            jnp.where(m, v, fill_v)
                    chid[pl.ds(slot * 16, 16)] = zero_v + ck
                    return jnp.minimum(nh + 1, CAPC)

                return lax.cond(c_s > 0, hit, lambda nh: nh, nh)

            for j in range(16):
                nh = one(j, nh)
            return nh

        nh = lax.fori_loop(0, NCHK // 16, grp_body, jnp.int32(0))
        nsl = nh * 4

        def count_ge(thr_s):
            thr = zero_v + thr_s

            def cb(sl, acc):
                vv = cand_v[pl.ds(sl * 16, 16)]
                return acc + jnp.where(vv >= thr, one_v, zero_v)

            acc = lax.fori_loop(0, nsl, cb, zero_v)
            return _bfly_sum(acc)

        def sb(_, carry):
            lo, hi = carry
            mid = lo + ((hi - lo) >> 1)
            pred = count_ge(lob_s + mid) >= need_s
            return (jnp.where(pred, mid, lo), jnp.where(pred, hi, mid))

        th_low, _ = lax.fori_loop(0, 16, sb, (jnp.int32(0), jnp.int32(65536)))
        thr = zero_v + (lob_s + th_low)

        def fb(h, carry):
            facc = zf
            for s4 in range(4):
                vv = cand_v[pl.ds((h * 4 + s4) * 16, 16)]
                mm = vv >= thr
                bb = jnp.where(vv >= 0, vv, INT_MIN - vv)
                ff = lax.bitcast_convert_type(bb, jnp.float32)
                facc = facc + jnp.where(mm, ff, zf)
            fs = _bfly_sum(facc)
            ck0 = chid[pl.ds(h * 16, 16)][0]
            cc_s = ck0 >> 4
            cls_smem[cc_s] = cls_smem[cc_s] + fs
            return carry

        lax.fori_loop(0, nh, fb, jnp.int32(0))

        for t in range(CPAD // 16):
            v = zf
            for j in range(16):
                sv = cls_smem[t * 16 + j]
                v = v + jnp.where(lane == j, zf + sv, zf)
            cls_sum[pl.ds(t * 16, 16)] = v
        pltpu.sync_copy(cls_sum, corr_hbm.at[pl.ds(r * CPAD, CPAD)])
        return carry

    lax.fori_loop(0, RPW, row_body, jnp.int32(0))


def _sc_correction(u1, lob1, need1, bc1):
    sc_fn = pl.kernel(
        _sc_body,
        mesh=plsc.VectorSubcoreMesh(core_axis_name="c", subcore_axis_name="s"),
        out_type=jax.ShapeDtypeStruct((B * CPAD,), jnp.float32),
        scratch_types=[
            pltpu.VMEM((NPAD,), jnp.int32),
            pltpu.VMEM((CAPC * CHK,), jnp.int32),
            pltpu.VMEM((CAPC * 16,), jnp.int32),
            pltpu.VMEM((NCHK,), jnp.int32),
            pltpu.VMEM((16,), jnp.int32),
            pltpu.VMEM((16,), jnp.int32),
            pltpu.VMEM((CPAD,), jnp.float32),
            pltpu.SMEM((CPAD,), jnp.float32),
        ],
    )
    return sc_fn(u1, lob1, need1, bc1)


def _k2b_body(defsum_ref, corr_ref, possum_ref, tgt_ref, loss_ref):
    logits = defsum_ref[...] + corr_ref[...]
    tgt = tgt_ref[...]
    c_iota = lax.broadcasted_iota(jnp.int32, (1, CPAD), 1)
    is_t = c_iota == tgt
    logits = logits + jnp.where(is_t, possum_ref[...], 0.0)
    logits = jnp.where(c_iota < C, logits, 0.0)
    lmask = 1.0 - (logits == 0.0).astype(jnp.float32)
    e = jnp.exp(logits) * lmask
    s = jnp.sum(jnp.where(c_iota < C, e, 0.0), axis=1, keepdims=True)
    e_t = jnp.sum(jnp.where(is_t, e, 0.0), axis=1, keepdims=True)
    predict_t = e_t / (1e-08 + s)
    rowloss = -jnp.log(predict_t + 1e-20)
    loss_ref[...] = jnp.reshape(jnp.sum(rowloss) * (1.0 / B), (1, 1))


@functools.partial(jax.jit, static_argnames=("interpret",))
def _run(x, target, proxies_padded, interpret=False):
    tgt2 = target.reshape(B, 1).astype(jnp.int32)
    u, possum = pl.pallas_call(
        _k1_body,
        grid=(NPAD // CT,),
        in_specs=[
            pl.BlockSpec((B, DIM), lambda cb: (0, 0)),
            pl.BlockSpec((DIM, CT), lambda cb: (0, cb)),
            pl.BlockSpec((B, 1), lambda cb: (0, 0)),
        ],
        out_specs=[
            pl.BlockSpec((B, CT), lambda cb: (0, cb)),
            pl.BlockSpec((B, 1), lambda cb: (0, 0)),
        ],
        out_shape=[
            jax.ShapeDtypeStruct((B, NPAD), jnp.int32),
            jax.ShapeDtypeStruct((B, 1), jnp.float32),
        ],
        interpret=interpret,
    )(x, proxies_padded, tgt2)

    lob, need, defsum, bc = pl.pallas_call(
        _k2a_body,
        grid=(B // BR,),
        in_specs=[
            pl.BlockSpec((BR, NPAD), lambda rb: (rb, 0)),
            pl.BlockSpec((BR, 1), lambda rb: (rb, 0)),
        ],
        out_specs=[
            pl.BlockSpec((BR, CPAD), lambda rb: (rb, 0)),
            pl.BlockSpec((BR, CPAD), lambda rb: (rb, 0)),
            pl.BlockSpec((BR, CPAD), lambda rb: (rb, 0)),
            pl.BlockSpec((BR, NCHK_PAD), lambda rb: (rb, 0)),
        ],
        out_shape=[
            jax.ShapeDtypeStruct((B, CPAD), jnp.int32),
            jax.ShapeDtypeStruct((B, CPAD), jnp.int32),
            jax.ShapeDtypeStruct((B, CPAD), jnp.float32),
            jax.ShapeDtypeStruct((B, NCHK_PAD), jnp.int32),
        ],
        interpret=interpret,
    )(u, tgt2)

    corr = _sc_correction(
        u.reshape(B * NPAD), lob.reshape(B * CPAD),
        need.reshape(B * CPAD), bc.reshape(B * NCHK_PAD),
    ).reshape(B, CPAD)

    loss = pl.pallas_call(
        _k2b_body,
        grid=(1,),
        in_specs=[
            pl.BlockSpec((B, CPAD), lambda i: (0, 0)),
            pl.BlockSpec((B, CPAD), lambda i: (0, 0)),
            pl.BlockSpec((B, 1), lambda i: (0, 0)),
            pl.BlockSpec((B, 1), lambda i: (0, 0)),
        ],
        out_specs=pl.BlockSpec((1, 1), lambda i: (0, 0)),
        out_shape=jax.ShapeDtypeStruct((1, 1), jnp.float32),
        interpret=interpret,
    )(defsum, corr, possum, tgt2)
    return loss[0, 0]


def kernel(input, target, Proxies, instance_label):
    p3 = Proxies.reshape(DIM, C, ALLNUM // C)
    p_pad = jnp.pad(p3, ((0, 0), (0, 0), (0, SEG - ALLNUM // C))).reshape(DIM, NPAD)
    loss = _run(input, target, p_pad)
    return (loss, jnp.array(0.0, dtype=jnp.float32))


# speedup vs baseline: 1.4502x; 1.4502x over previous
import functools
import math

import jax
import jax.numpy as jnp
from jax import lax
from jax.experimental import pallas as pl
from jax.experimental.pallas import tpu as pltpu
from jax.experimental.pallas import tpu_sc as plsc

C = 100
ALLNUM = 100000
DIM = 64
B = 1024
TOPK = 5000
SEG = 1024
NPAD = C * SEG
CT = 2048
BR = 32
CPAD = 128
INT_MIN = -2147483648
NW = 32
RPW = B // NW
CHK = 64
NCHK = NPAD // CHK
NCHK_PAD = 2048
CAPC = 64
BWIDTH = 256
NBITS = 8

_DN = lax.GatherDimensionNumbers(
    offset_dims=(), collapsed_slice_dims=(0,), start_index_map=(0,))


def _key_from_boosted(boosted):
    b = lax.bitcast_convert_type(boosted, jnp.int32)
    return jnp.where(b >= 0, b, INT_MIN - b)


def _val_from_key(u):
    b = jnp.where(u >= 0, u, INT_MIN - u)
    return lax.bitcast_convert_type(b, jnp.float32)


def _k1_body(x_ref, p_ref, tgt_ref, u_ref, possum_ref):
    cb = pl.program_id(0)
    pt = p_ref[...]
    n2 = jnp.sum(pt * pt, axis=0, keepdims=True)
    invn = 1.0 / jnp.maximum(jnp.sqrt(n2), 1e-12)
    sim = jnp.dot(x_ref[...], pt,
                  preferred_element_type=jnp.float32) * invn
    j = lax.broadcasted_iota(jnp.int32, (1, CT), 1)
    cls = cb * (CT // SEG) + (j // SEG)
    ispad = (j % SEG) >= (ALLNUM // C)
    tgt = tgt_ref[...]
    pos = (cls == tgt) & jnp.logical_not(ispad)
    boosted = sim + 1000.0 * pos.astype(jnp.float32)
    u = _key_from_boosted(boosted)
    u_ref[...] = jnp.where(ispad, INT_MIN, u)

    contrib = jnp.sum(jnp.where(pos, sim, 0.0), axis=1, keepdims=True)

    @pl.when(cb == 0)
    def _():
        possum_ref[...] = jnp.zeros_like(possum_ref)

    possum_ref[...] += contrib


def _k2a_body(u_ref, tgt_ref, lob_ref, need_ref, defsum_ref, bc_ref):
    u = u_ref[...]

    def body(_, carry):
        lo, hi, cnt_hi = carry
        mid = lo + ((hi - lo) >> 1)
        cnt = jnp.sum((u >= (mid << 8)).astype(jnp.int32), axis=1,
                      keepdims=True)
        pred = cnt >= TOPK
        lo = jnp.where(pred, mid, lo)
        hi = jnp.where(pred, hi, mid)
        cnt_hi = jnp.where(pred, cnt_hi, cnt)
        return lo, hi, cnt_hi

    lo0 = jnp.full((BR, 1), -(1 << 23), jnp.int32)
    hi0 = jnp.full((BR, 1), 1 << 23, jnp.int32)
    ch0 = jnp.zeros((BR, 1), jnp.int32)
    th24, _, cnt_above = lax.fori_loop(0, 24, body, (lo0, hi0, ch0))

    lob = th24 << 8
    hib = (th24 + 1) << 8
    lob_ref[...] = lob + jnp.zeros((BR, CPAD), jnp.int32)
    need_ref[...] = (TOPK - cnt_above) + jnp.zeros((BR, CPAD), jnp.int32)

    j = lax.broadcasted_iota(jnp.int32, (1, NPAD), 1)
    cls = j // SEG
    tgt = tgt_ref[...]
    seldef = (u >= hib) & (cls != tgt)
    vals = jnp.where(seldef, _val_from_key(u), 0.0)
    dsum = jnp.sum(vals.reshape(BR, C, SEG), axis=2)
    defsum_ref[...] = jnp.concatenate(
        [dsum, jnp.zeros((BR, CPAD - C), jnp.float32)], axis=1)

    inb = (u >= lob) & (u < hib)
    mi = jnp.where(inb, 1, 0).astype(jnp.int32)
    bc = jnp.sum(mi.reshape(BR, NCHK, CHK), axis=2)
    bc_ref[...] = jnp.concatenate(
        [bc, jnp.zeros((BR, NCHK_PAD - NCHK), jnp.int32)], axis=1)


def _bfly_sum(x):
    idx = lax.iota(jnp.int32, 16)
    for s in (8, 4, 2, 1):
        perm = idx ^ s
        x = x + lax.gather(x, perm[:, None], _DN, slice_sizes=(1,),
                           mode=lax.GatherScatterMode.PROMISE_IN_BOUNDS)
    return x[0]


def _sc_body(u_hbm, lob_hbm, need_hbm, bc_hbm, corr_hbm,
             chunk_buf, cand_v, chid, bc_buf, lob_buf, need_buf, cls_sum,
             cls_smem):
    wid = lax.axis_index("s") * 2 + lax.axis_index("c")
    base = wid * RPW
    lane = lax.iota(jnp.int32, 16)

    def row_body(i, carry):
        r = base + i
        pltpu.sync_copy(bc_hbm.at[pl.ds(r * NCHK_PAD, NCHK)], bc_buf)
        pltpu.sync_copy(lob_hbm.at[pl.ds(r * CPAD, 16)], lob_buf)
        pltpu.sync_copy(need_hbm.at[pl.ds(r * CPAD, 16)], need_buf)
        lob_v = lob_buf[pl.ds(0, 16)]
        lob_s = lob_v[0]
        need_s = need_buf[pl.ds(0, 16)][0]
        width_v = jnp.full((16,), BWIDTH, jnp.uint32)
        fill_v = jnp.full((16,), INT_MIN, jnp.int32)
        zero_v = jnp.full((16,), 0, jnp.int32)
        one_v = jnp.full((16,), 1, jnp.int32)
        zf = jnp.zeros((16,), jnp.float32)
        for t in range(CPAD):
            cls_smem[t] = 0.0

        def grp_body(g, nh):
            cv = bc_buf[pl.ds(g * 16, 16)]
            tot = _bfly_sum(cv)

            def scan_group(nh):
                def one(j, nh):
                    c_s = cv[j]

                    def hit(nh):
                        slot = jnp.minimum(nh, CAPC - 1)
                        ck = g * 16 + j
                        pltpu.sync_copy(
                            u_hbm.at[pl.ds(r * NPAD + ck * CHK, CHK)],
                            chunk_buf)
                        for s4 in range(CHK // 16):
                            v = chunk_buf[pl.ds(s4 * 16, 16)]
                            d = lax.bitcast_convert_type(v - lob_v,
                                                         jnp.uint32)
                            m = d < width_v
                            cand_v[pl.ds((slot * 4 + s4) * 16, 16)] = \
                                jnp.where(m, v, fill_v)
                        chid[pl.ds(slot * 16, 16)] = zero_v + ck
                        return jnp.minimum(nh + 1, CAPC)

                    return lax.cond(c_s > 0, hit, lambda nh: nh, nh)

                for j in range(16):
                    nh = one(j, nh)
                return nh

            return lax.cond(tot > 0, scan_group, lambda nh: nh, nh)

        nh = lax.fori_loop(0, NCHK // 16, grp_body, jnp.int32(0))
        nsl = nh * 4

        def count_ge(thr_s):
            thr = zero_v + thr_s

            def cb(sl, acc):
                vv = cand_v[pl.ds(sl * 16, 16)]
                return acc + jnp.where(vv >= thr, one_v, zero_v)

            acc = lax.fori_loop(0, nsl, cb, zero_v)
            return _bfly_sum(acc)

        def sb(_, carry):
            lo, hi = carry
            mid = lo + ((hi - lo) >> 1)
            pred = count_ge(lob_s + mid) >= need_s
            return (jnp.where(pred, mid, lo), jnp.where(pred, hi, mid))

        th_low, _ = lax.fori_loop(0, NBITS, sb,
                                  (jnp.int32(0), jnp.int32(BWIDTH)))
        thr = zero_v + (lob_s + th_low)

        def fb(h, carry):
            facc = zf
            for s4 in range(4):
                vv = cand_v[pl.ds((h * 4 + s4) * 16, 16)]
                mm = vv >= thr
                bb = jnp.where(vv >= 0, vv, INT_MIN - vv)
                ff = lax.bitcast_convert_type(bb, jnp.float32)
                facc = facc + jnp.where(mm, ff, zf)
            fs = _bfly_sum(facc)
            ck0 = chid[pl.ds(h * 16, 16)][0]
            cc_s = ck0 >> 4
            cls_smem[cc_s] = cls_smem[cc_s] + fs
            return carry

        lax.fori_loop(0, nh, fb, jnp.int32(0))

        for t in range(CPAD // 16):
            v = zf
            for j in range(16):
                sv = cls_smem[t * 16 + j]
                v = v + jnp.where(lane == j, zf + sv, zf)
            cls_sum[pl.ds(t * 16, 16)] = v
        pltpu.sync_copy(cls_sum, corr_hbm.at[pl.ds(r * CPAD, CPAD)])
        return carry

    lax.fori_loop(0, RPW, row_body, jnp.int32(0))


def _sc_correction(u1, lob1, need1, bc1):
    sc_fn = pl.kernel(
        _sc_body,
        mesh=plsc.VectorSubcoreMesh(core_axis_name="c", subcore_axis_name="s"),
        out_type=jax.ShapeDtypeStruct((B * CPAD,), jnp.float32),
        scratch_types=[
            pltpu.VMEM((CHK,), jnp.int32),
            pltpu.VMEM((CAPC * CHK,), jnp.int32),
            pltpu.VMEM((CAPC * 16,), jnp.int32),
            pltpu.VMEM((NCHK,), jnp.int32),
            pltpu.VMEM((16,), jnp.int32),
            pltpu.VMEM((16,), jnp.int32),
            pltpu.VMEM((CPAD,), jnp.float32),
            pltpu.SMEM((CPAD,), jnp.float32),
        ],
    )
    return sc_fn(u1, lob1, need1, bc1)


def _k2b_body(defsum_ref, corr_ref, possum_ref, tgt_ref, loss_ref):
    logits = defsum_ref[...] + corr_ref[...]
    tgt = tgt_ref[...]
    c_iota = lax.broadcasted_iota(jnp.int32, (1, CPAD), 1)
    is_t = c_iota == tgt
    logits = logits + jnp.where(is_t, possum_ref[...], 0.0)
    logits = jnp.where(c_iota < C, logits, 0.0)
    lmask = 1.0 - (logits == 0.0).astype(jnp.float32)
    e = jnp.exp(logits) * lmask
    s = jnp.sum(jnp.where(c_iota < C, e, 0.0), axis=1, keepdims=True)
    e_t = jnp.sum(jnp.where(is_t, e, 0.0), axis=1, keepdims=True)
    predict_t = e_t / (1e-08 + s)
    rowloss = -jnp.log(predict_t + 1e-20)
    loss_ref[...] = jnp.reshape(jnp.sum(rowloss) * (1.0 / B), (1, 1))


@functools.partial(jax.jit, static_argnames=("interpret",))
def _run(x, target, proxies_padded, interpret=False):
    tgt2 = target.reshape(B, 1).astype(jnp.int32)
    u, possum = pl.pallas_call(
        _k1_body,
        grid=(NPAD // CT,),
        in_specs=[
            pl.BlockSpec((B, DIM), lambda cb: (0, 0)),
            pl.BlockSpec((DIM, CT), lambda cb: (0, cb)),
            pl.BlockSpec((B, 1), lambda cb: (0, 0)),
        ],
        out_specs=[
            pl.BlockSpec((B, CT), lambda cb: (0, cb)),
            pl.BlockSpec((B, 1), lambda cb: (0, 0)),
        ],
        out_shape=[
            jax.ShapeDtypeStruct((B, NPAD), jnp.int32),
            jax.ShapeDtypeStruct((B, 1), jnp.float32),
        ],
        interpret=interpret,
    )(x, proxies_padded, tgt2)

    lob, need, defsum, bc = pl.pallas_call(
        _k2a_body,
        grid=(B // BR,),
        in_specs=[
            pl.BlockSpec((BR, NPAD), lambda rb: (rb, 0)),
            pl.BlockSpec((BR, 1), lambda rb: (rb, 0)),
        ],
        out_specs=[
            pl.BlockSpec((BR, CPAD), lambda rb: (rb, 0)),
            pl.BlockSpec((BR, CPAD), lambda rb: (rb, 0)),
            pl.BlockSpec((BR, CPAD), lambda rb: (rb, 0)),
            pl.BlockSpec((BR, NCHK_PAD), lambda rb: (rb, 0)),
        ],
        out_shape=[
            jax.ShapeDtypeStruct((B, CPAD), jnp.int32),
            jax.ShapeDtypeStruct((B, CPAD), jnp.int32),
            jax.ShapeDtypeStruct((B, CPAD), jnp.float32),
            jax.ShapeDtypeStruct((B, NCHK_PAD), jnp.int32),
        ],
        interpret=interpret,
    )(u, tgt2)

    corr = _sc_correction(
        u.reshape(B * NPAD), lob.reshape(B * CPAD),
        need.reshape(B * CPAD), bc.reshape(B * NCHK_PAD),
    ).reshape(B, CPAD)

    loss = pl.pallas_call(
        _k2b_body,
        grid=(1,),
        in_specs=[
            pl.BlockSpec((B, CPAD), lambda i: (0, 0)),
            pl.BlockSpec((B, CPAD), lambda i: (0, 0)),
            pl.BlockSpec((B, 1), lambda i: (0, 0)),
            pl.BlockSpec((B, 1), lambda i: (0, 0)),
        ],
        out_specs=pl.BlockSpec((1, 1), lambda i: (0, 0)),
        out_shape=jax.ShapeDtypeStruct((1, 1), jnp.float32),
        interpret=interpret,
    )(defsum, corr, possum, tgt2)
    return loss[0, 0]


def kernel(input, target, Proxies, instance_label):
    p3 = Proxies.reshape(DIM, C, ALLNUM // C)
    p_pad = jnp.pad(p3, ((0, 0), (0, 0), (0, SEG - ALLNUM // C))).reshape(DIM, NPAD)
    loss = _run(input, target, p_pad)
    return (loss, jnp.array(0.0, dtype=jnp.float32))


# speedup vs baseline: 1.4654x; 1.0104x over previous
import functools
import math

import jax
import jax.numpy as jnp
from jax import lax
from jax.experimental import pallas as pl
from jax.experimental.pallas import tpu as pltpu
from jax.experimental.pallas import tpu_sc as plsc

C = 100
ALLNUM = 100000
DIM = 64
B = 1024
TOPK = 5000
SEG = 1024
NPAD = C * SEG
CT = 2048
BR = 32
CPAD = 128
INT_MIN = -2147483648
NW = 32
RPW = B // NW
CHK = 64
NCHK = NPAD // CHK
NCHK_PAD = 2048
CAPC = 64
BWIDTH = 256
NBITS = 8

_DN = lax.GatherDimensionNumbers(
    offset_dims=(), collapsed_slice_dims=(0,), start_index_map=(0,))


def _key_from_boosted(boosted):
    b = lax.bitcast_convert_type(boosted, jnp.int32)
    return jnp.where(b >= 0, b, INT_MIN - b)


def _val_from_key(u):
    b = jnp.where(u >= 0, u, INT_MIN - u)
    return lax.bitcast_convert_type(b, jnp.float32)


def _k1_body(x_ref, p_ref, tgt_ref, u_ref, possum_ref):
    cb = pl.program_id(0)
    pt = p_ref[...]
    n2 = jnp.sum(pt * pt, axis=0, keepdims=True)
    invn = 1.0 / jnp.maximum(jnp.sqrt(n2), 1e-12)
    sim = jnp.dot(x_ref[...], pt,
                  preferred_element_type=jnp.float32) * invn
    j = lax.broadcasted_iota(jnp.int32, (1, CT), 1)
    cls = cb * (CT // SEG) + (j // SEG)
    ispad = (j % SEG) >= (ALLNUM // C)
    tgt = tgt_ref[...]
    pos = (cls == tgt) & jnp.logical_not(ispad)
    boosted = sim + 1000.0 * pos.astype(jnp.float32)
    u = _key_from_boosted(boosted)
    u_ref[...] = jnp.where(ispad, INT_MIN, u)

    contrib = jnp.sum(jnp.where(pos, sim, 0.0), axis=1, keepdims=True)

    @pl.when(cb == 0)
    def _():
        possum_ref[...] = jnp.zeros_like(possum_ref)

    possum_ref[...] += contrib


def _k2a_body(u_ref, tgt_ref, lob_ref, need_ref, defsum_ref, bc_ref):
    u = u_ref[...]

    def body(_, carry):
        lo, hi, cnt_hi = carry
        mid = lo + ((hi - lo) >> 1)
        cnt = jnp.sum((u >= (mid << 8)).astype(jnp.int32), axis=1,
                      keepdims=True)
        pred = cnt >= TOPK
        lo = jnp.where(pred, mid, lo)
        hi = jnp.where(pred, hi, mid)
        cnt_hi = jnp.where(pred, cnt_hi, cnt)
        return lo, hi, cnt_hi

    lo0 = jnp.full((BR, 1), -(1 << 23), jnp.int32)
    hi0 = jnp.full((BR, 1), 1 << 23, jnp.int32)
    ch0 = jnp.zeros((BR, 1), jnp.int32)
    th24, _, cnt_above = lax.fori_loop(0, 24, body, (lo0, hi0, ch0))

    lob = th24 << 8
    hib = (th24 + 1) << 8
    lob_ref[...] = lob + jnp.zeros((BR, CPAD), jnp.int32)
    need_ref[...] = (TOPK - cnt_above) + jnp.zeros((BR, CPAD), jnp.int32)

    j = lax.broadcasted_iota(jnp.int32, (1, NPAD), 1)
    cls = j // SEG
    tgt = tgt_ref[...]
    seldef = (u >= hib) & (cls != tgt)
    vals = jnp.where(seldef, _val_from_key(u), 0.0)
    dsum = jnp.sum(vals.reshape(BR, C, SEG), axis=2)
    defsum_ref[...] = jnp.concatenate(
        [dsum, jnp.zeros((BR, CPAD - C), jnp.float32)], axis=1)

    inb = (u >= lob) & (u < hib)
    mi = jnp.where(inb, 1, 0).astype(jnp.int32)
    bc = jnp.sum(mi.reshape(BR, NCHK, CHK), axis=2)
    bc_ref[...] = jnp.concatenate(
        [bc, jnp.zeros((BR, NCHK_PAD - NCHK), jnp.int32)], axis=1)


def _bfly_sum(x):
    idx = lax.iota(jnp.int32, 16)
    for s in (8, 4, 2, 1):
        perm = idx ^ s
        x = x + lax.gather(x, perm[:, None], _DN, slice_sizes=(1,),
                           mode=lax.GatherScatterMode.PROMISE_IN_BOUNDS)
    return x[0]


def _sc_body(u_hbm, lob_hbm, need_hbm, bc_hbm, corr_hbm,
             chunk_buf, cand_v, chid, bc_buf, lob_buf, need_buf, corr_buf,
             cls_smem):
    wid = lax.axis_index("s") * 2 + lax.axis_index("c")
    base = wid * RPW
    lane = lax.iota(jnp.int32, 16)
    pltpu.sync_copy(bc_hbm.at[pl.ds(base * NCHK_PAD, RPW * NCHK_PAD)], bc_buf)
    pltpu.sync_copy(lob_hbm.at[pl.ds(base * CPAD, RPW * CPAD)], lob_buf)
    pltpu.sync_copy(need_hbm.at[pl.ds(base * CPAD, RPW * CPAD)], need_buf)

    def row_body(i, carry):
        r = base + i
        lob_v = lob_buf[pl.ds(i * CPAD, 16)]
        lob_s = lob_v[0]
        need_s = need_buf[pl.ds(i * CPAD, 16)][0]
        width_v = jnp.full((16,), BWIDTH, jnp.uint32)
        fill_v = jnp.full((16,), INT_MIN, jnp.int32)
        zero_v = jnp.full((16,), 0, jnp.int32)
        one_v = jnp.full((16,), 1, jnp.int32)
        zf = jnp.zeros((16,), jnp.float32)
        for t in range(CPAD):
            cls_smem[t] = 0.0

        def grp_body(g, nh):
            cv = bc_buf[pl.ds(i * NCHK_PAD + g * 16, 16)]
            tot = _bfly_sum(cv)

            def scan_group(nh):
                def one(j, nh):
                    c_s = cv[j]

                    def hit(nh):
                        slot = jnp.minimum(nh, CAPC - 1)
                        ck = g * 16 + j
                        pltpu.sync_copy(
                            u_hbm.at[pl.ds(r * NPAD + ck * CHK, CHK)],
                            chunk_buf)
                        for s4 in range(CHK // 16):
                            v = chunk_buf[pl.ds(s4 * 16, 16)]
                            d = lax.bitcast_convert_type(v - lob_v,
                                                         jnp.uint32)
                            m = d < width_v
                            cand_v[pl.ds((slot * 4 + s4) * 16, 16)] = \
                                jnp.where(m, v, fill_v)
                        chid[pl.ds(slot * 16, 16)] = zero_v + ck
                        return jnp.minimum(nh + 1, CAPC)

                    return lax.cond(c_s > 0, hit, lambda nh: nh, nh)

                for j in range(16):
                    nh = one(j, nh)
                return nh

            return lax.cond(tot > 0, scan_group, lambda nh: nh, nh)

        nh = lax.fori_loop(0, NCHK // 16, grp_body, jnp.int32(0))
        nsl = nh * 4

        def count_ge(thr_s):
            thr = zero_v + thr_s

            def cb(sl, acc):
                vv = cand_v[pl.ds(sl * 16, 16)]
                return acc + jnp.where(vv >= thr, one_v, zero_v)

            acc = lax.fori_loop(0, nsl, cb, zero_v)
            return _bfly_sum(acc)

        def sb(_, carry):
            lo, hi = carry
            mid = lo + ((hi - lo) >> 1)
            pred = count_ge(lob_s + mid) >= need_s
            return (jnp.where(pred, mid, lo), jnp.where(pred, hi, mid))

        th_low, _ = lax.fori_loop(0, NBITS, sb,
                                  (jnp.int32(0), jnp.int32(BWIDTH)))
        thr = zero_v + (lob_s + th_low)

        def fb(h, carry):
            facc = zf
            for s4 in range(4):
                vv = cand_v[pl.ds((h * 4 + s4) * 16, 16)]
                mm = vv >= thr
                bb = jnp.where(vv >= 0, vv, INT_MIN - vv)
                ff = lax.bitcast_convert_type(bb, jnp.float32)
                facc = facc + jnp.where(mm, ff, zf)
            fs = _bfly_sum(facc)
            ck0 = chid[pl.ds(h * 16, 16)][0]
            cc_s = ck0 >> 4
            cls_smem[cc_s] = cls_smem[cc_s] + fs
            return carry

        lax.fori_loop(0, nh, fb, jnp.int32(0))

        for t in range(CPAD // 16):
            v = zf
            for j in range(16):
                sv = cls_smem[t * 16 + j]
                v = v + jnp.where(lane == j, zf + sv, zf)
            corr_buf[pl.ds(i * CPAD + t * 16, 16)] = v
        return carry

    lax.fori_loop(0, RPW, row_body, jnp.int32(0))
    pltpu.sync_copy(corr_buf, corr_hbm.at[pl.ds(base * CPAD, RPW * CPAD)])


def _sc_correction(u1, lob1, need1, bc1):
    sc_fn = pl.kernel(
        _sc_body,
        mesh=plsc.VectorSubcoreMesh(core_axis_name="c", subcore_axis_name="s"),
        out_type=jax.ShapeDtypeStruct((B * CPAD,), jnp.float32),
        scratch_types=[
            pltpu.VMEM((CHK,), jnp.int32),
            pltpu.VMEM((CAPC * CHK,), jnp.int32),
            pltpu.VMEM((CAPC * 16,), jnp.int32),
            pltpu.VMEM((RPW * NCHK_PAD,), jnp.int32),
            pltpu.VMEM((RPW * CPAD,), jnp.int32),
            pltpu.VMEM((RPW * CPAD,), jnp.int32),
            pltpu.VMEM((RPW * CPAD,), jnp.float32),
            pltpu.SMEM((CPAD,), jnp.float32),
        ],
    )
    return sc_fn(u1, lob1, need1, bc1)


def _k2b_body(defsum_ref, corr_ref, possum_ref, tgt_ref, loss_ref):
    logits = defsum_ref[...] + corr_ref[...]
    tgt = tgt_ref[...]
    c_iota = lax.broadcasted_iota(jnp.int32, (1, CPAD), 1)
    is_t = c_iota == tgt
    logits = logits + jnp.where(is_t, possum_ref[...], 0.0)
    logits = jnp.where(c_iota < C, logits, 0.0)
    lmask = 1.0 - (logits == 0.0).astype(jnp.float32)
    e = jnp.exp(logits) * lmask
    s = jnp.sum(jnp.where(c_iota < C, e, 0.0), axis=1, keepdims=True)
    e_t = jnp.sum(jnp.where(is_t, e, 0.0), axis=1, keepdims=True)
    predict_t = e_t / (1e-08 + s)
    rowloss = -jnp.log(predict_t + 1e-20)
    loss_ref[...] = jnp.reshape(jnp.sum(rowloss) * (1.0 / B), (1, 1))


@functools.partial(jax.jit, static_argnames=("interpret",))
def _run(x, target, proxies_padded, interpret=False):
    tgt2 = target.reshape(B, 1).astype(jnp.int32)
    u, possum = pl.pallas_call(
        _k1_body,
        grid=(NPAD // CT,),
        in_specs=[
            pl.BlockSpec((B, DIM), lambda cb: (0, 0)),
            pl.BlockSpec((DIM, CT), lambda cb: (0, cb)),
            pl.BlockSpec((B, 1), lambda cb: (0, 0)),
        ],
        out_specs=[
            pl.BlockSpec((B, CT), lambda cb: (0, cb)),
            pl.BlockSpec((B, 1), lambda cb: (0, 0)),
        ],
        out_shape=[
            jax.ShapeDtypeStruct((B, NPAD), jnp.int32),
            jax.ShapeDtypeStruct((B, 1), jnp.float32),
        ],
        interpret=interpret,
    )(x, proxies_padded, tgt2)

    lob, need, defsum, bc = pl.pallas_call(
        _k2a_body,
        grid=(B // BR,),
        in_specs=[
            pl.BlockSpec((BR, NPAD), lambda rb: (rb, 0)),
            pl.BlockSpec((BR, 1), lambda rb: (rb, 0)),
        ],
        out_specs=[
            pl.BlockSpec((BR, CPAD), lambda rb: (rb, 0)),
            pl.BlockSpec((BR, CPAD), lambda rb: (rb, 0)),
            pl.BlockSpec((BR, CPAD), lambda rb: (rb, 0)),
            pl.BlockSpec((BR, NCHK_PAD), lambda rb: (rb, 0)),
        ],
        out_shape=[
            jax.ShapeDtypeStruct((B, CPAD), jnp.int32),
            jax.ShapeDtypeStruct((B, CPAD), jnp.int32),
            jax.ShapeDtypeStruct((B, CPAD), jnp.float32),
            jax.ShapeDtypeStruct((B, NCHK_PAD), jnp.int32),
        ],
        interpret=interpret,
    )(u, tgt2)

    corr = _sc_correction(
        u.reshape(B * NPAD), lob.reshape(B * CPAD),
        need.reshape(B * CPAD), bc.reshape(B * NCHK_PAD),
    ).reshape(B, CPAD)

    loss = pl.pallas_call(
        _k2b_body,
        grid=(1,),
        in_specs=[
            pl.BlockSpec((B, CPAD), lambda i: (0, 0)),
            pl.BlockSpec((B, CPAD), lambda i: (0, 0)),
            pl.BlockSpec((B, 1), lambda i: (0, 0)),
            pl.BlockSpec((B, 1), lambda i: (0, 0)),
        ],
        out_specs=pl.BlockSpec((1, 1), lambda i: (0, 0)),
        out_shape=jax.ShapeDtypeStruct((1, 1), jnp.float32),
        interpret=interpret,
    )(defsum, corr, possum, tgt2)
    return loss[0, 0]


def kernel(input, target, Proxies, instance_label):
    p3 = Proxies.reshape(DIM, C, ALLNUM // C)
    p_pad = jnp.pad(p3, ((0, 0), (0, 0), (0, SEG - ALLNUM // C))).reshape(DIM, NPAD)
    loss = _run(input, target, p_pad)
    return (loss, jnp.array(0.0, dtype=jnp.float32))


# speedup vs baseline: 1.5777x; 1.0767x over previous
import functools
import math

import jax
import jax.numpy as jnp
from jax import lax
from jax.experimental import pallas as pl
from jax.experimental.pallas import tpu as pltpu
from jax.experimental.pallas import tpu_sc as plsc

C = 100
ALLNUM = 100000
DIM = 64
B = 1024
TOPK = 5000
SEG = 1024
NPAD = C * SEG
CT = 2048
BR = 32
CPAD = 128
INT_MIN = -2147483648
NW = 32
RPW = B // NW
CHK = 64
NCHK = NPAD // CHK
NCHK_PAD = 2048
CAPC = 64
BWIDTH = 256
NBITS = 8

_DN = lax.GatherDimensionNumbers(
    offset_dims=(), collapsed_slice_dims=(0,), start_index_map=(0,))


def _key_from_boosted(boosted):
    b = lax.bitcast_convert_type(boosted, jnp.int32)
    return jnp.where(b >= 0, b, INT_MIN - b)


def _val_from_key(u):
    b = jnp.where(u >= 0, u, INT_MIN - u)
    return lax.bitcast_convert_type(b, jnp.float32)


def _k1_body(x_ref, p_ref, tgt_ref, u_ref, possum_ref):
    cb = pl.program_id(0)
    pt = p_ref[...]
    n2 = jnp.sum(pt * pt, axis=0, keepdims=True)
    invn = 1.0 / jnp.maximum(jnp.sqrt(n2), 1e-12)
    sim = jnp.dot(x_ref[...], pt,
                  preferred_element_type=jnp.float32) * invn
    j = lax.broadcasted_iota(jnp.int32, (1, CT), 1)
    cls = cb * (CT // SEG) + (j // SEG)
    ispad = (j % SEG) >= (ALLNUM // C)
    tgt = tgt_ref[...]
    pos = (cls == tgt) & jnp.logical_not(ispad)
    boosted = sim + 1000.0 * pos.astype(jnp.float32)
    u = _key_from_boosted(boosted)
    u_ref[...] = jnp.where(ispad, INT_MIN, u)

    contrib = jnp.sum(jnp.where(pos, sim, 0.0), axis=1, keepdims=True)

    @pl.when(cb == 0)
    def _():
        possum_ref[...] = jnp.zeros_like(possum_ref)

    possum_ref[...] += contrib


def _k2a_body(u_ref, tgt_ref, lob_ref, need_ref, defsum_ref, bc_ref):
    u = u_ref[...]

    def body(_, carry):
        lo, hi, cnt_hi = carry
        mid = lo + ((hi - lo) >> 1)
        cnt = jnp.sum((u >= (mid << 8)).astype(jnp.int32), axis=1,
                      keepdims=True)
        pred = cnt >= TOPK
        lo = jnp.where(pred, mid, lo)
        hi = jnp.where(pred, hi, mid)
        cnt_hi = jnp.where(pred, cnt_hi, cnt)
        return lo, hi, cnt_hi

    lo0 = jnp.full((BR, 1), -(1 << 23), jnp.int32)
    hi0 = jnp.full((BR, 1), 1 << 23, jnp.int32)
    ch0 = jnp.zeros((BR, 1), jnp.int32)
    th24, _, cnt_above = lax.fori_loop(0, 24, body, (lo0, hi0, ch0))

    lob = th24 << 8
    hib = (th24 + 1) << 8
    lob_ref[...] = lob + jnp.zeros((BR, CPAD), jnp.int32)
    need_ref[...] = (TOPK - cnt_above) + jnp.zeros((BR, CPAD), jnp.int32)

    j = lax.broadcasted_iota(jnp.int32, (1, NPAD), 1)
    cls = j // SEG
    tgt = tgt_ref[...]
    seldef = (u >= hib) & (cls != tgt)
    vals = jnp.where(seldef, _val_from_key(u), 0.0)
    dsum = jnp.sum(vals.reshape(BR, C, SEG), axis=2)
    defsum_ref[...] = jnp.concatenate(
        [dsum, jnp.zeros((BR, CPAD - C), jnp.float32)], axis=1)

    inb = (u >= lob) & (u < hib)
    mi = jnp.where(inb, 1, 0).astype(jnp.int32)
    bc = jnp.sum(mi.reshape(BR, NCHK, CHK), axis=2)
    bc_ref[...] = jnp.concatenate(
        [bc, jnp.zeros((BR, NCHK_PAD - NCHK), jnp.int32)], axis=1)


def _bfly_sum(x):
    idx = lax.iota(jnp.int32, 16)
    for s in (8, 4, 2, 1):
        perm = idx ^ s
        x = x + lax.gather(x, perm[:, None], _DN, slice_sizes=(1,),
                           mode=lax.GatherScatterMode.PROMISE_IN_BOUNDS)
    return x[0]


def _sc_body(u_hbm, lob_hbm, need_hbm, bc_hbm, corr_hbm,
             chunk_buf, cand_v, chid, bc_buf, lob_buf, need_buf, corr_buf,
             cls_smem):
    wid = lax.axis_index("s") * 2 + lax.axis_index("c")
    base = wid * RPW
    lane = lax.iota(jnp.int32, 16)
    pltpu.sync_copy(bc_hbm.at[pl.ds(base * NCHK_PAD, RPW * NCHK_PAD)], bc_buf)
    pltpu.sync_copy(lob_hbm.at[pl.ds(base * CPAD, RPW * CPAD)], lob_buf)
    pltpu.sync_copy(need_hbm.at[pl.ds(base * CPAD, RPW * CPAD)], need_buf)

    def row_body(i, carry):
        r = base + i
        lob_v = lob_buf[pl.ds(i * CPAD, 16)]
        lob_s = lob_v[0]
        need_s = need_buf[pl.ds(i * CPAD, 16)][0]
        width_v = jnp.full((16,), BWIDTH, jnp.uint32)
        fill_v = jnp.full((16,), INT_MIN, jnp.int32)
        zero_v = jnp.full((16,), 0, jnp.int32)
        one_v = jnp.full((16,), 1, jnp.int32)
        zf = jnp.zeros((16,), jnp.float32)
        for t in range(CPAD):
            cls_smem[t] = 0.0

        def grp_body(g, nh):
            cv = bc_buf[pl.ds(i * NCHK_PAD + g * 16, 16)]
            tot = _bfly_sum(cv)

            def scan_group(nh):
                def one(j, nh):
                    c_s = cv[j]

                    def hit(nh):
                        slot = jnp.minimum(nh, CAPC - 1)
                        ck = g * 16 + j
                        r8 = pl.multiple_of((r >> 3) << 3, 8)
                        sub = r & 7
                        half = (ck & 1) * CHK
                        col = pl.multiple_of((ck >> 1) << 7, 128)
                        pltpu.sync_copy(
                            u_hbm.at[pl.ds(r8, 8), pl.ds(col, 128)],
                            chunk_buf)
                        for sub_s in range(8):
                            @pl.when(sub == sub_s)
                            def _():
                                for s4 in range(CHK // 16):
                                    v = chunk_buf[sub_s,
                                                  pl.ds(half + s4 * 16, 16)]
                                    d = lax.bitcast_convert_type(
                                        v - lob_v, jnp.uint32)
                                    m = d < width_v
                                    cand_v[pl.ds((slot * 4 + s4) * 16, 16)] = \
                                        jnp.where(m, v, fill_v)
                        chid[pl.ds(slot * 16, 16)] = zero_v + ck
                        return jnp.minimum(nh + 1, CAPC)

                    return lax.cond(c_s > 0, hit, lambda nh: nh, nh)

                for j in range(16):
                    nh = one(j, nh)
                return nh

            return lax.cond(tot > 0, scan_group, lambda nh: nh, nh)

        nh = lax.fori_loop(0, NCHK // 16, grp_body, jnp.int32(0))
        nsl = nh * 4

        def count_ge(thr_s):
            thr = zero_v + thr_s

            def cb(sl, acc):
                vv = cand_v[pl.ds(sl * 16, 16)]
                return acc + jnp.where(vv >= thr, one_v, zero_v)

            acc = lax.fori_loop(0, nsl, cb, zero_v)
            return _bfly_sum(acc)

        def sb(_, carry):
            lo, hi = carry
            mid = lo + ((hi - lo) >> 1)
            pred = count_ge(lob_s + mid) >= need_s
            return (jnp.where(pred, mid, lo), jnp.where(pred, hi, mid))

        th_low, _ = lax.fori_loop(0, NBITS, sb,
                                  (jnp.int32(0), jnp.int32(BWIDTH)))
        thr = zero_v + (lob_s + th_low)

        def fb(h, carry):
            facc = zf
            for s4 in range(4):
                vv = cand_v[pl.ds((h * 4 + s4) * 16, 16)]
                mm = vv >= thr
                bb = jnp.where(vv >= 0, vv, INT_MIN - vv)
                ff = lax.bitcast_convert_type(bb, jnp.float32)
                facc = facc + jnp.where(mm, ff, zf)
            fs = _bfly_sum(facc)
            ck0 = chid[pl.ds(h * 16, 16)][0]
            cc_s = ck0 >> 4
            cls_smem[cc_s] = cls_smem[cc_s] + fs
            return carry

        lax.fori_loop(0, nh, fb, jnp.int32(0))

        for t in range(CPAD // 16):
            v = zf
            for j in range(16):
                sv = cls_smem[t * 16 + j]
                v = v + jnp.where(lane == j, zf + sv, zf)
            corr_buf[pl.ds(i * CPAD + t * 16, 16)] = v
        return carry

    lax.fori_loop(0, RPW, row_body, jnp.int32(0))
    pltpu.sync_copy(corr_buf, corr_hbm.at[pl.ds(base * CPAD, RPW * CPAD)])


def _sc_correction(u1, lob1, need1, bc1):
    sc_fn = pl.kernel(
        _sc_body,
        mesh=plsc.VectorSubcoreMesh(core_axis_name="c", subcore_axis_name="s"),
        out_type=jax.ShapeDtypeStruct((B * CPAD,), jnp.float32),
        scratch_types=[
            pltpu.VMEM((8, 128), jnp.int32),
            pltpu.VMEM((CAPC * CHK,), jnp.int32),
            pltpu.VMEM((CAPC * 16,), jnp.int32),
            pltpu.VMEM((RPW * NCHK_PAD,), jnp.int32),
            pltpu.VMEM((RPW * CPAD,), jnp.int32),
            pltpu.VMEM((RPW * CPAD,), jnp.int32),
            pltpu.VMEM((RPW * CPAD,), jnp.float32),
            pltpu.SMEM((CPAD,), jnp.float32),
        ],
    )
    return sc_fn(u1, lob1, need1, bc1)


def _k2b_body(defsum_ref, corr_ref, possum_ref, tgt_ref, loss_ref):
    logits = defsum_ref[...] + corr_ref[...]
    tgt = tgt_ref[...]
    c_iota = lax.broadcasted_iota(jnp.int32, (1, CPAD), 1)
    is_t = c_iota == tgt
    logits = logits + jnp.where(is_t, possum_ref[...], 0.0)
    logits = jnp.where(c_iota < C, logits, 0.0)
    lmask = 1.0 - (logits == 0.0).astype(jnp.float32)
    e = jnp.exp(logits) * lmask
    s = jnp.sum(jnp.where(c_iota < C, e, 0.0), axis=1, keepdims=True)
    e_t = jnp.sum(jnp.where(is_t, e, 0.0), axis=1, keepdims=True)
    predict_t = e_t / (1e-08 + s)
    rowloss = -jnp.log(predict_t + 1e-20)
    loss_ref[...] = jnp.reshape(jnp.sum(rowloss) * (1.0 / B), (1, 1))


@functools.partial(jax.jit, static_argnames=("interpret",))
def _run(x, target, proxies_padded, interpret=False):
    tgt2 = target.reshape(B, 1).astype(jnp.int32)
    u, possum = pl.pallas_call(
        _k1_body,
        grid=(NPAD // CT,),
        in_specs=[
            pl.BlockSpec((B, DIM), lambda cb: (0, 0)),
            pl.BlockSpec((DIM, CT), lambda cb: (0, cb)),
            pl.BlockSpec((B, 1), lambda cb: (0, 0)),
        ],
        out_specs=[
            pl.BlockSpec((B, CT), lambda cb: (0, cb)),
            pl.BlockSpec((B, 1), lambda cb: (0, 0)),
        ],
        out_shape=[
            jax.ShapeDtypeStruct((B, NPAD), jnp.int32),
            jax.ShapeDtypeStruct((B, 1), jnp.float32),
        ],
        interpret=interpret,
    )(x, proxies_padded, tgt2)

    lob, need, defsum, bc = pl.pallas_call(
        _k2a_body,
        grid=(B // BR,),
        in_specs=[
            pl.BlockSpec((BR, NPAD), lambda rb: (rb, 0)),
            pl.BlockSpec((BR, 1), lambda rb: (rb, 0)),
        ],
        out_specs=[
            pl.BlockSpec((BR, CPAD), lambda rb: (rb, 0)),
            pl.BlockSpec((BR, CPAD), lambda rb: (rb, 0)),
            pl.BlockSpec((BR, CPAD), lambda rb: (rb, 0)),
            pl.BlockSpec((BR, NCHK_PAD), lambda rb: (rb, 0)),
        ],
        out_shape=[
            jax.ShapeDtypeStruct((B, CPAD), jnp.int32),
            jax.ShapeDtypeStruct((B, CPAD), jnp.int32),
            jax.ShapeDtypeStruct((B, CPAD), jnp.float32),
            jax.ShapeDtypeStruct((B, NCHK_PAD), jnp.int32),
        ],
        interpret=interpret,
    )(u, tgt2)

    corr = _sc_correction(
        u, lob.reshape(B * CPAD),
        need.reshape(B * CPAD), bc.reshape(B * NCHK_PAD),
    ).reshape(B, CPAD)

    loss = pl.pallas_call(
        _k2b_body,
        grid=(1,),
        in_specs=[
            pl.BlockSpec((B, CPAD), lambda i: (0, 0)),
            pl.BlockSpec((B, CPAD), lambda i: (0, 0)),
            pl.BlockSpec((B, 1), lambda i: (0, 0)),
            pl.BlockSpec((B, 1), lambda i: (0, 0)),
        ],
        out_specs=pl.BlockSpec((1, 1), lambda i: (0, 0)),
        out_shape=jax.ShapeDtypeStruct((1, 1), jnp.float32),
        interpret=interpret,
    )(defsum, corr, possum, tgt2)
    return loss[0, 0]


def kernel(input, target, Proxies, instance_label):
    p3 = Proxies.reshape(DIM, C, ALLNUM // C)
    p_pad = jnp.pad(p3, ((0, 0), (0, 0), (0, SEG - ALLNUM // C))).reshape(DIM, NPAD)
    loss = _run(input, target, p_pad)
    return (loss, jnp.array(0.0, dtype=jnp.float32))
